# Initial kernel scaffold; baseline (speedup 1.0000x reference)
import jax
import jax.numpy as jnp
from jax.experimental import pallas as pl


def kernel(h, edge_index, W1, a1s, a1d, W2, a2s, a2d, W3, a3s, a3d, Wi, Whh, bi, bh, W4, b4):
    raise NotImplementedError("write your pallas kernel here")


# speedup vs baseline: 1.7447x; 1.7447x over previous
import functools
import jax
import jax.numpy as jnp
from jax.experimental import pallas as pl
from jax.experimental.pallas import tpu as pltpu

N = 10000
E = 320000
TN = 400


def _proj_body(h_ref, w_ref, a_ref, wh_ref, ea_ref):
    wh = jnp.dot(h_ref[...], w_ref[...], preferred_element_type=jnp.float32)
    wh_ref[...] = wh
    ea_ref[...] = jnp.dot(wh, a_ref[...], preferred_element_type=jnp.float32)


def _proj(h, Wcat, Acat):
    Din = h.shape[1]
    HDH = Wcat.shape[1]
    A2 = Acat.shape[1]
    grid = (N // TN,)
    return pl.pallas_call(
        _proj_body,
        grid=grid,
        in_specs=[
            pl.BlockSpec((TN, Din), lambda i: (i, 0)),
            pl.BlockSpec((Din, HDH), lambda i: (0, 0)),
            pl.BlockSpec((HDH, A2), lambda i: (0, 0)),
        ],
        out_specs=[
            pl.BlockSpec((TN, HDH), lambda i: (i, 0)),
            pl.BlockSpec((TN, A2), lambda i: (i, 0)),
        ],
        out_shape=[
            jax.ShapeDtypeStruct((N, HDH), jnp.float32),
            jax.ShapeDtypeStruct((N, A2), jnp.float32),
        ],
    )(h, Wcat, Acat)


def _gru_body(x_ref, h_ref, wi_ref, wh_ref, b_ref, o_ref):
    dh = h_ref.shape[1]
    gi = jnp.dot(x_ref[...], wi_ref[...], preferred_element_type=jnp.float32)
    gh = jnp.dot(h_ref[...], wh_ref[...], preferred_element_type=jnp.float32)
    gi = gi + b_ref[0, :][None, :]
    gh = gh + b_ref[1, :][None, :]
    ir, iz, inn = gi[:, :dh], gi[:, dh:2 * dh], gi[:, 2 * dh:]
    hr, hz, hn = gh[:, :dh], gh[:, dh:2 * dh], gh[:, 2 * dh:]
    r = jax.nn.sigmoid(ir + hr)
    z = jax.nn.sigmoid(iz + hz)
    n = jnp.tanh(inn + r * hn)
    o_ref[...] = (1.0 - z) * n + z * h_ref[...]


def _gru(x, hprev, Wi, Whh, bi, bh):
    dh = hprev.shape[1]
    b = jnp.stack([bi, bh])
    return pl.pallas_call(
        _gru_body,
        grid=(N // TN,),
        in_specs=[
            pl.BlockSpec((TN, dh), lambda i: (i, 0)),
            pl.BlockSpec((TN, dh), lambda i: (i, 0)),
            pl.BlockSpec((dh, 3 * dh), lambda i: (0, 0)),
            pl.BlockSpec((dh, 3 * dh), lambda i: (0, 0)),
            pl.BlockSpec((2, 3 * dh), lambda i: (0, 0)),
        ],
        out_specs=pl.BlockSpec((TN, dh), lambda i: (i, 0)),
        out_shape=jax.ShapeDtypeStruct((N, dh), jnp.float32),
    )(x, hprev, Wi, Whh, b)


def _final_body(h_ref, w_ref, b_ref, o_ref):
    y = jnp.dot(h_ref[...], w_ref[...], preferred_element_type=jnp.float32)
    o_ref[...] = jax.nn.sigmoid(y + b_ref[...])


def _final(h3, W4, b4):
    dh = h3.shape[1]
    do = W4.shape[1]
    return pl.pallas_call(
        _final_body,
        grid=(N // TN,),
        in_specs=[
            pl.BlockSpec((TN, dh), lambda i: (i, 0)),
            pl.BlockSpec((dh, do), lambda i: (0, 0)),
            pl.BlockSpec((1, do), lambda i: (0, 0)),
        ],
        out_specs=pl.BlockSpec((TN, do), lambda i: (i, 0)),
        out_shape=jax.ShapeDtypeStruct((N, do), jnp.float32),
    )(h3, W4, b4[None, :])


def _edge_phase(Wh, ea, src, dst, H, DH):
    es = ea[:, :H]
    ed = ea[:, H:]
    e = jax.nn.leaky_relu(es[src] + ed[dst], 0.2)
    ex = jnp.exp(e)
    denom = jax.ops.segment_sum(ex, dst, num_segments=N)
    msg = Wh[src].reshape(E, H, DH) * ex[:, :, None]
    unnorm = jax.ops.segment_sum(msg.reshape(E, H * DH), dst, num_segments=N)
    agg = (unnorm.reshape(N, H, DH) / (denom[:, :, None] + 1e-9)).mean(axis=1)
    return jax.nn.elu(agg)


def kernel(h, edge_index, W1, a1s, a1d, W2, a2s, a2d, W3, a3s, a3d, Wi, Whh, bi, bh, W4, b4):
    src = edge_index[0]
    dst = edge_index[1]

    def acat(a_s, a_d):
        H, DH = a_s.shape
        eye = jnp.eye(H, dtype=jnp.float32)
        As = (a_s[:, None, :] * eye[:, :, None]).transpose(0, 2, 1).reshape(H * DH, H)
        Ad = (a_d[:, None, :] * eye[:, :, None]).transpose(0, 2, 1).reshape(H * DH, H)
        return jnp.concatenate([As, Ad], axis=1)

    H1, DIN, DH = W1.shape
    Wc1 = W1.transpose(1, 0, 2).reshape(DIN, H1 * DH)
    Wh1, ea1 = _proj(h, Wc1, acat(a1s, a1d))
    h1 = _edge_phase(Wh1, ea1, src, dst, H1, DH)

    H2 = W2.shape[0]
    Wc2 = W2.transpose(1, 0, 2).reshape(DH, H2 * DH)
    Wh2, ea2 = _proj(h1, Wc2, acat(a2s, a2d))
    a2 = _edge_phase(Wh2, ea2, src, dst, H2, DH)
    h2 = _gru(a2, h1, Wi, Whh, bi, bh)

    H3 = W3.shape[0]
    Wc3 = W3.transpose(1, 0, 2).reshape(DH, H3 * DH)
    Wh3, ea3 = _proj(h2, Wc3, acat(a3s, a3d))
    a3 = _edge_phase(Wh3, ea3, src, dst, H3, DH)
    h3 = _gru(a3, h2, Wi, Whh, bi, bh)

    return _final(h3, W4, b4)


# speedup vs baseline: 2.5425x; 1.4573x over previous
import functools
import jax
import jax.numpy as jnp
from jax import lax
from jax.experimental import pallas as pl
from jax.experimental.pallas import tpu as pltpu
from jax.experimental.pallas import tpu_sc as plsc

N = 10000
E = 320000
TN = 400


def _proj_body(h_ref, w_ref, a_ref, wh_ref, ea_ref):
    wh = jnp.dot(h_ref[...], w_ref[...], preferred_element_type=jnp.float32)
    wh_ref[...] = wh
    ea_ref[...] = jnp.dot(wh, a_ref[...], preferred_element_type=jnp.float32)


def _proj(h, Wcat, Acat):
    Din = h.shape[1]
    HDH = Wcat.shape[1]
    A2 = Acat.shape[1]
    grid = (N // TN,)
    return pl.pallas_call(
        _proj_body,
        grid=grid,
        in_specs=[
            pl.BlockSpec((TN, Din), lambda i: (i, 0)),
            pl.BlockSpec((Din, HDH), lambda i: (0, 0)),
            pl.BlockSpec((HDH, A2), lambda i: (0, 0)),
        ],
        out_specs=[
            pl.BlockSpec((TN, HDH), lambda i: (i, 0)),
            pl.BlockSpec((TN, A2), lambda i: (i, 0)),
        ],
        out_shape=[
            jax.ShapeDtypeStruct((N, HDH), jnp.float32),
            jax.ShapeDtypeStruct((N, A2), jnp.float32),
        ],
    )(h, Wcat, Acat)


def _gru_body(x_ref, h_ref, wi_ref, wh_ref, b_ref, o_ref):
    dh = h_ref.shape[1]
    gi = jnp.dot(x_ref[...], wi_ref[...], preferred_element_type=jnp.float32)
    gh = jnp.dot(h_ref[...], wh_ref[...], preferred_element_type=jnp.float32)
    gi = gi + b_ref[0, :][None, :]
    gh = gh + b_ref[1, :][None, :]
    ir, iz, inn = gi[:, :dh], gi[:, dh:2 * dh], gi[:, 2 * dh:]
    hr, hz, hn = gh[:, :dh], gh[:, dh:2 * dh], gh[:, 2 * dh:]
    r = jax.nn.sigmoid(ir + hr)
    z = jax.nn.sigmoid(iz + hz)
    n = jnp.tanh(inn + r * hn)
    o_ref[...] = (1.0 - z) * n + z * h_ref[...]


def _gru(x, hprev, Wi, Whh, bi, bh):
    dh = hprev.shape[1]
    b = jnp.stack([bi, bh])
    return pl.pallas_call(
        _gru_body,
        grid=(N // TN,),
        in_specs=[
            pl.BlockSpec((TN, dh), lambda i: (i, 0)),
            pl.BlockSpec((TN, dh), lambda i: (i, 0)),
            pl.BlockSpec((dh, 3 * dh), lambda i: (0, 0)),
            pl.BlockSpec((dh, 3 * dh), lambda i: (0, 0)),
            pl.BlockSpec((2, 3 * dh), lambda i: (0, 0)),
        ],
        out_specs=pl.BlockSpec((TN, dh), lambda i: (i, 0)),
        out_shape=jax.ShapeDtypeStruct((N, dh), jnp.float32),
    )(x, hprev, Wi, Whh, b)


def _final_body(h_ref, w_ref, b_ref, o_ref):
    y = jnp.dot(h_ref[...], w_ref[...], preferred_element_type=jnp.float32)
    o_ref[...] = jax.nn.sigmoid(y + b_ref[...])


def _final(h3, W4, b4):
    dh = h3.shape[1]
    do = W4.shape[1]
    return pl.pallas_call(
        _final_body,
        grid=(N // TN,),
        in_specs=[
            pl.BlockSpec((TN, dh), lambda i: (i, 0)),
            pl.BlockSpec((dh, do), lambda i: (0, 0)),
            pl.BlockSpec((1, do), lambda i: (0, 0)),
        ],
        out_specs=pl.BlockSpec((TN, do), lambda i: (i, 0)),
        out_shape=jax.ShapeDtypeStruct((N, do), jnp.float32),
    )(h3, W4, b4[None, :])


_NHALF = N // 2
_NPAD = 5120
_DUMP = 5100
_CHUNK = 400


def _sc_pass1(ea_t, edge_index, H):
    epc = E // 32
    nchunks = epc // _CHUNK
    hgroups = [(h0, min(2, H - h0)) for h0 in range(0, H, 2)]
    mesh = plsc.VectorSubcoreMesh(core_axis_name="c", subcore_axis_name="s")

    @functools.partial(
        pl.kernel, mesh=mesh,
        compiler_params=pltpu.CompilerParams(needs_layout_passes=False),
        out_type=[
            jax.ShapeDtypeStruct((H * E,), jnp.float32),
            jax.ShapeDtypeStruct((32 * H * N,), jnp.float32),
        ],
        scratch_types=[
            pltpu.VMEM((4 * N,), jnp.float32),
            pltpu.VMEM((2 * N,), jnp.float32),
            pltpu.VMEM((_CHUNK,), jnp.int32),
            pltpu.VMEM((_CHUNK,), jnp.int32),
            pltpu.VMEM((2 * _CHUNK,), jnp.float32),
        ],
    )
    def k(ea_hbm, src_hbm, dst_hbm, ex_hbm, pd_hbm, tab, dn, srcv, dstv, exst):
        wid = lax.axis_index("s") * 2 + lax.axis_index("c")
        ebase = wid * epc
        zero16 = jnp.zeros((16,), jnp.float32)
        for (h0, nh) in hgroups:
            pltpu.sync_copy(ea_hbm.at[pl.ds(h0 * N, nh * N)], tab.at[pl.ds(0, nh * N)])
            pltpu.sync_copy(ea_hbm.at[pl.ds((H + h0) * N, nh * N)],
                            tab.at[pl.ds(2 * N, nh * N)])

            def zdn(i, _):
                dn[pl.ds(i * 16, 16)] = zero16
                return _
            lax.fori_loop(0, 2 * N // 16, zdn, None)

            def chunk_body(ci, _):
                eoff = ebase + ci * _CHUNK
                pltpu.sync_copy(src_hbm.at[pl.ds(eoff, _CHUNK)], srcv)
                pltpu.sync_copy(dst_hbm.at[pl.ds(eoff, _CHUNK)], dstv)

                def group(b, _g):
                    s16 = srcv[pl.ds(b * 16, 16)]
                    d16 = dstv[pl.ds(b * 16, 16)]
                    for hh in range(nh):
                        es16 = plsc.load_gather(tab, [s16 + hh * N])
                        ed16 = plsc.load_gather(tab, [d16 + (2 + hh) * N])
                        e16 = es16 + ed16
                        e16 = jnp.where(e16 > 0, e16, 0.2 * e16)
                        x16 = jnp.exp(e16)
                        exst[pl.ds(hh * _CHUNK + b * 16, 16)] = x16
                        plsc.addupdate_scatter(dn, [d16 + hh * N], x16)
                    return _g
                lax.fori_loop(0, _CHUNK // 16, group, None)
                for hh in range(nh):
                    pltpu.sync_copy(exst.at[pl.ds(hh * _CHUNK, _CHUNK)],
                                    ex_hbm.at[pl.ds((h0 + hh) * E + eoff, _CHUNK)])
                return _
            lax.fori_loop(0, nchunks, chunk_body, None)
            pltpu.sync_copy(dn.at[pl.ds(0, nh * N)],
                            pd_hbm.at[pl.ds(wid * H * N + h0 * N, nh * N)])

    src_a = edge_index[0].astype(jnp.int32)
    dst_a = edge_index[1].astype(jnp.int32)
    return k(ea_t.reshape(-1), src_a, dst_a)


def _psum_body(p_ref, o_ref):
    o_ref[...] = jnp.sum(p_ref[...], axis=0, keepdims=True)


def _psum(p):
    X = p.shape[1]
    return pl.pallas_call(
        _psum_body,
        out_shape=jax.ShapeDtypeStruct((1, X), jnp.float32),
    )(p)


def _sc_alpha(ex, denom, dst_a, H):
    epc = E // 32
    nchunks = epc // _CHUNK
    mesh = plsc.VectorSubcoreMesh(core_axis_name="c", subcore_axis_name="s")

    @functools.partial(
        pl.kernel, mesh=mesh,
        compiler_params=pltpu.CompilerParams(needs_layout_passes=False),
        out_type=jax.ShapeDtypeStruct((H * E,), jnp.float32),
        scratch_types=[
            pltpu.VMEM((H * N,), jnp.float32),
            pltpu.VMEM((_CHUNK,), jnp.int32),
            pltpu.VMEM((H * _CHUNK,), jnp.float32),
        ],
    )
    def k(ex_hbm, den_hbm, dst_hbm, al_hbm, dtab, dstv, exv):
        wid = lax.axis_index("s") * 2 + lax.axis_index("c")
        ebase = wid * epc
        pltpu.sync_copy(den_hbm, dtab)

        def chunk_body(ci, _):
            eoff = ebase + ci * _CHUNK
            pltpu.sync_copy(dst_hbm.at[pl.ds(eoff, _CHUNK)], dstv)
            for h in range(H):
                pltpu.sync_copy(ex_hbm.at[pl.ds(h * E + eoff, _CHUNK)],
                                exv.at[pl.ds(h * _CHUNK, _CHUNK)])

            def group(b, _g):
                d16 = dstv[pl.ds(b * 16, 16)]
                for h in range(H):
                    x16 = exv[pl.ds(h * _CHUNK + b * 16, 16)]
                    dn16 = plsc.load_gather(dtab, [d16 + h * N])
                    exv[pl.ds(h * _CHUNK + b * 16, 16)] = (
                        x16 / (dn16 + 1e-9) * (1.0 / H))
                return _g
            lax.fori_loop(0, _CHUNK // 16, group, None)
            for h in range(H):
                pltpu.sync_copy(exv.at[pl.ds(h * _CHUNK, _CHUNK)],
                                al_hbm.at[pl.ds(h * E + eoff, _CHUNK)])
            return _
        lax.fori_loop(0, nchunks, chunk_body, None)

    return k(ex, denom, dst_a)


def _sc_pass2(wh_flat, alpha, edge_index, H, DH):
    epc = E // 16
    nchunks = epc // _CHUNK
    mesh = plsc.VectorSubcoreMesh(core_axis_name="c", subcore_axis_name="s")

    @functools.partial(
        pl.kernel, mesh=mesh,
        compiler_params=pltpu.CompilerParams(needs_layout_passes=False),
        out_type=jax.ShapeDtypeStruct((2, _NPAD, DH), jnp.float32),
        scratch_types=[
            pltpu.VMEM((_CHUNK,), jnp.int32),
            pltpu.VMEM((_CHUNK,), jnp.int32),
            pltpu.VMEM((H * _CHUNK,), jnp.float32),
            pltpu.VMEM((16 * H,), jnp.int32),
            pltpu.VMEM((16 * H, DH), jnp.float32),
            pltpu.VMEM((80, DH), jnp.float32),
            pltpu.VMEM((80,), jnp.int32),
            pltpu.VMEM((8, DH), jnp.float32),
            pltpu.VMEM_SHARED((_NPAD, DH), jnp.float32),
            pltpu.SemaphoreType.DMA,
        ],
    )
    def k2(wh_hbm, al_hbm, src_hbm, dst_hbm, out_hbm,
           srcv, dstv, exv, idxb, rows, stage, ridx, zbuf, aggs, sem):
        cid = lax.axis_index("c")
        sid = lax.axis_index("s")
        zero16 = jnp.zeros((16,), jnp.float32)
        for r in range(8):
            for c in range(DH // 16):
                zbuf[r, pl.ds(c * 16, 16)] = zero16

        def zrow(i, _):
            pltpu.sync_copy(zbuf, aggs.at[pl.ds(sid * (_NPAD // 16) + i * 8, 8)])
            return _
        lax.fori_loop(0, _NPAD // 16 // 8, zrow, None)
        plsc.subcore_barrier()

        nlo = cid * _NHALF

        def chunk_body(ci, _):
            eoff = sid * epc + ci * _CHUNK
            pltpu.sync_copy(src_hbm.at[pl.ds(eoff, _CHUNK)], srcv)
            pltpu.sync_copy(dst_hbm.at[pl.ds(eoff, _CHUNK)], dstv)
            for h in range(H):
                pltpu.sync_copy(al_hbm.at[pl.ds(h * E + eoff, _CHUNK)],
                                exv.at[pl.ds(h * _CHUNK, _CHUNK)])

            def group(b, _g):
                sb = (b % 5) * 16
                s16 = srcv[pl.ds(b * 16, 16)]
                d16 = dstv[pl.ds(b * 16, 16)]
                for h in range(H):
                    idxb[pl.ds(h * 16, 16)] = s16 * H + h
                cp = pltpu.async_copy(wh_hbm.at[idxb], rows, sem)
                local = d16 - nlo
                fmask = (local < 0) | (local >= _NHALF)
                ridx[pl.ds(sb, 16)] = jnp.where(fmask, _DUMP, local)
                alphas = []
                for h in range(H):
                    a16 = exv[pl.ds(h * _CHUNK + b * 16, 16)]
                    alphas.append(jnp.where(fmask, 0.0, a16))
                cp.wait()
                for e in range(16):
                    als = [alphas[h][e] for h in range(H)]
                    for c in range(DH // 16):
                        acc = rows[e, pl.ds(c * 16, 16)] * als[0]
                        for h in range(1, H):
                            acc = acc + rows[h * 16 + e, pl.ds(c * 16, 16)] * als[h]
                        stage[sb + e, pl.ds(c * 16, 16)] = acc

                @pl.when(b % 5 == 4)
                def _flush():
                    pltpu.sync_copy(stage, aggs.at[ridx], add=True)
                return _g
            lax.fori_loop(0, _CHUNK // 16, group, None)
            return _
        lax.fori_loop(0, nchunks, chunk_body, None)
        plsc.subcore_barrier()
        rows_per_tile = _NPAD // 16
        pltpu.sync_copy(aggs.at[pl.ds(sid * rows_per_tile, rows_per_tile)],
                        out_hbm.at[cid, pl.ds(sid * rows_per_tile, rows_per_tile)])

    src_a = edge_index[0].astype(jnp.int32)
    dst_a = edge_index[1].astype(jnp.int32)
    return k2(wh_flat, alpha, src_a, dst_a)


def _elu_body(x_ref, o_ref):
    x = x_ref[...]
    o_ref[...] = jnp.where(x > 0, x, jnp.exp(x) - 1.0)


def _elu(x):
    dh = x.shape[1]
    return pl.pallas_call(
        _elu_body,
        grid=(N // TN,),
        in_specs=[pl.BlockSpec((TN, dh), lambda i: (i, 0))],
        out_specs=pl.BlockSpec((TN, dh), lambda i: (i, 0)),
        out_shape=jax.ShapeDtypeStruct((N, dh), jnp.float32),
    )(x)


def _edge_phase(Wh, ea, edge_index, H, DH):
    ea_t = ea.T
    ex, pd = _sc_pass1(ea_t, edge_index, H)
    denom = _psum(pd.reshape(32, H * N)).reshape(H * N)
    alpha = _sc_alpha(ex, denom, edge_index[1].astype(jnp.int32), H)
    src = edge_index[0]
    dst = edge_index[1]
    al = alpha.reshape(H, E).T
    msg = Wh[src].reshape(E, H, DH) * al[:, :, None]
    agg = jax.ops.segment_sum(msg.reshape(E, H * DH), dst, num_segments=N)
    agg = agg.reshape(N, H, DH).sum(axis=1)
    return _elu(agg)


def kernel(h, edge_index, W1, a1s, a1d, W2, a2s, a2d, W3, a3s, a3d, Wi, Whh, bi, bh, W4, b4):
    src = edge_index[0]
    dst = edge_index[1]

    def acat(a_s, a_d):
        H, DH = a_s.shape
        eye = jnp.eye(H, dtype=jnp.float32)
        As = (a_s[:, None, :] * eye[:, :, None]).transpose(0, 2, 1).reshape(H * DH, H)
        Ad = (a_d[:, None, :] * eye[:, :, None]).transpose(0, 2, 1).reshape(H * DH, H)
        return jnp.concatenate([As, Ad], axis=1)

    H1, DIN, DH = W1.shape
    Wc1 = W1.transpose(1, 0, 2).reshape(DIN, H1 * DH)
    Wh1, ea1 = _proj(h, Wc1, acat(a1s, a1d))
    h1 = _edge_phase(Wh1, ea1, edge_index, H1, DH)

    H2 = W2.shape[0]
    Wc2 = W2.transpose(1, 0, 2).reshape(DH, H2 * DH)
    Wh2, ea2 = _proj(h1, Wc2, acat(a2s, a2d))
    a2 = _edge_phase(Wh2, ea2, edge_index, H2, DH)
    h2 = _gru(a2, h1, Wi, Whh, bi, bh)

    H3 = W3.shape[0]
    Wc3 = W3.transpose(1, 0, 2).reshape(DH, H3 * DH)
    Wh3, ea3 = _proj(h2, Wc3, acat(a3s, a3d))
    a3 = _edge_phase(Wh3, ea3, edge_index, H3, DH)
    h3 = _gru(a3, h2, Wi, Whh, bi, bh)

    return _final(h3, W4, b4)


# speedup vs baseline: 2.6531x; 1.0435x over previous
import functools
import jax
import jax.numpy as jnp
from jax import lax
from jax.experimental import pallas as pl
from jax.experimental.pallas import tpu as pltpu
from jax.experimental.pallas import tpu_sc as plsc

N = 10000
E = 320000
TN = 400


def _proj_body(h_ref, w_ref, a_ref, wh_ref, ea_ref):
    wh = jnp.dot(h_ref[...], w_ref[...], preferred_element_type=jnp.float32)
    wh_ref[...] = wh
    ea_ref[...] = jnp.dot(wh, a_ref[...], preferred_element_type=jnp.float32)


def _proj(h, Wcat, Acat):
    Din = h.shape[1]
    HDH = Wcat.shape[1]
    A2 = Acat.shape[1]
    grid = (N // TN,)
    return pl.pallas_call(
        _proj_body,
        grid=grid,
        in_specs=[
            pl.BlockSpec((TN, Din), lambda i: (i, 0)),
            pl.BlockSpec((Din, HDH), lambda i: (0, 0)),
            pl.BlockSpec((HDH, A2), lambda i: (0, 0)),
        ],
        out_specs=[
            pl.BlockSpec((TN, HDH), lambda i: (i, 0)),
            pl.BlockSpec((TN, A2), lambda i: (i, 0)),
        ],
        out_shape=[
            jax.ShapeDtypeStruct((N, HDH), jnp.float32),
            jax.ShapeDtypeStruct((N, A2), jnp.float32),
        ],
    )(h, Wcat, Acat)


def _gru_body(x_ref, h_ref, wi_ref, wh_ref, b_ref, o_ref):
    dh = h_ref.shape[1]
    gi = jnp.dot(x_ref[...], wi_ref[...], preferred_element_type=jnp.float32)
    gh = jnp.dot(h_ref[...], wh_ref[...], preferred_element_type=jnp.float32)
    gi = gi + b_ref[0, :][None, :]
    gh = gh + b_ref[1, :][None, :]
    ir, iz, inn = gi[:, :dh], gi[:, dh:2 * dh], gi[:, 2 * dh:]
    hr, hz, hn = gh[:, :dh], gh[:, dh:2 * dh], gh[:, 2 * dh:]
    r = jax.nn.sigmoid(ir + hr)
    z = jax.nn.sigmoid(iz + hz)
    n = jnp.tanh(inn + r * hn)
    o_ref[...] = (1.0 - z) * n + z * h_ref[...]


def _gru(x, hprev, Wi, Whh, bi, bh):
    dh = hprev.shape[1]
    b = jnp.stack([bi, bh])
    return pl.pallas_call(
        _gru_body,
        grid=(N // TN,),
        in_specs=[
            pl.BlockSpec((TN, dh), lambda i: (i, 0)),
            pl.BlockSpec((TN, dh), lambda i: (i, 0)),
            pl.BlockSpec((dh, 3 * dh), lambda i: (0, 0)),
            pl.BlockSpec((dh, 3 * dh), lambda i: (0, 0)),
            pl.BlockSpec((2, 3 * dh), lambda i: (0, 0)),
        ],
        out_specs=pl.BlockSpec((TN, dh), lambda i: (i, 0)),
        out_shape=jax.ShapeDtypeStruct((N, dh), jnp.float32),
    )(x, hprev, Wi, Whh, b)


def _final_body(h_ref, w_ref, b_ref, o_ref):
    y = jnp.dot(h_ref[...], w_ref[...], preferred_element_type=jnp.float32)
    o_ref[...] = jax.nn.sigmoid(y + b_ref[...])


def _final(h3, W4, b4):
    dh = h3.shape[1]
    do = W4.shape[1]
    return pl.pallas_call(
        _final_body,
        grid=(N // TN,),
        in_specs=[
            pl.BlockSpec((TN, dh), lambda i: (i, 0)),
            pl.BlockSpec((dh, do), lambda i: (0, 0)),
            pl.BlockSpec((1, do), lambda i: (0, 0)),
        ],
        out_specs=pl.BlockSpec((TN, do), lambda i: (i, 0)),
        out_shape=jax.ShapeDtypeStruct((N, do), jnp.float32),
    )(h3, W4, b4[None, :])


_NHALF = N // 2
_NPAD = 5120
_DUMP = 5100
_CHUNK = 400


def _sc_pass1(ea_t, edge_index, H):
    epc = E // 32
    nchunks = epc // _CHUNK
    hgroups = [(h0, min(2, H - h0)) for h0 in range(0, H, 2)]
    mesh = plsc.VectorSubcoreMesh(core_axis_name="c", subcore_axis_name="s")

    @functools.partial(
        pl.kernel, mesh=mesh,
        compiler_params=pltpu.CompilerParams(needs_layout_passes=False),
        out_type=[
            jax.ShapeDtypeStruct((H * E,), jnp.float32),
            jax.ShapeDtypeStruct((32 * H * N,), jnp.float32),
        ],
        scratch_types=[
            pltpu.VMEM((4 * N,), jnp.float32),
            pltpu.VMEM((2 * N,), jnp.float32),
            pltpu.VMEM((_CHUNK,), jnp.int32),
            pltpu.VMEM((_CHUNK,), jnp.int32),
            pltpu.VMEM((2 * _CHUNK,), jnp.float32),
        ],
    )
    def k(ea_hbm, src_hbm, dst_hbm, ex_hbm, pd_hbm, tab, dn, srcv, dstv, exst):
        wid = lax.axis_index("s") * 2 + lax.axis_index("c")
        ebase = wid * epc
        zero16 = jnp.zeros((16,), jnp.float32)
        for (h0, nh) in hgroups:
            pltpu.sync_copy(ea_hbm.at[pl.ds(h0 * N, nh * N)], tab.at[pl.ds(0, nh * N)])
            pltpu.sync_copy(ea_hbm.at[pl.ds((H + h0) * N, nh * N)],
                            tab.at[pl.ds(2 * N, nh * N)])

            def zdn(i, _):
                dn[pl.ds(i * 16, 16)] = zero16
                return _
            lax.fori_loop(0, 2 * N // 16, zdn, None)

            def chunk_body(ci, _):
                eoff = ebase + ci * _CHUNK
                pltpu.sync_copy(src_hbm.at[pl.ds(eoff, _CHUNK)], srcv)
                pltpu.sync_copy(dst_hbm.at[pl.ds(eoff, _CHUNK)], dstv)

                def group(b, _g):
                    s16 = srcv[pl.ds(b * 16, 16)]
                    d16 = dstv[pl.ds(b * 16, 16)]
                    for hh in range(nh):
                        es16 = plsc.load_gather(tab, [s16 + hh * N])
                        ed16 = plsc.load_gather(tab, [d16 + (2 + hh) * N])
                        e16 = es16 + ed16
                        e16 = jnp.where(e16 > 0, e16, 0.2 * e16)
                        x16 = jnp.exp(e16)
                        exst[pl.ds(hh * _CHUNK + b * 16, 16)] = x16
                        plsc.addupdate_scatter(dn, [d16 + hh * N], x16)
                    return _g
                lax.fori_loop(0, _CHUNK // 16, group, None)
                for hh in range(nh):
                    pltpu.sync_copy(exst.at[pl.ds(hh * _CHUNK, _CHUNK)],
                                    ex_hbm.at[pl.ds((h0 + hh) * E + eoff, _CHUNK)])
                return _
            lax.fori_loop(0, nchunks, chunk_body, None)
            pltpu.sync_copy(dn.at[pl.ds(0, nh * N)],
                            pd_hbm.at[pl.ds(wid * H * N + h0 * N, nh * N)])

    src_a = edge_index[0].astype(jnp.int32)
    dst_a = edge_index[1].astype(jnp.int32)
    return k(ea_t.reshape(-1), src_a, dst_a)


def _psum_body(p_ref, o_ref):
    o_ref[...] = jnp.sum(p_ref[...], axis=0, keepdims=True)


def _psum(p):
    X = p.shape[1]
    return pl.pallas_call(
        _psum_body,
        out_shape=jax.ShapeDtypeStruct((1, X), jnp.float32),
    )(p)


def _sc_alpha(ex, denom, dst_a, H):
    epc = E // 32
    nchunks = epc // _CHUNK
    mesh = plsc.VectorSubcoreMesh(core_axis_name="c", subcore_axis_name="s")

    @functools.partial(
        pl.kernel, mesh=mesh,
        compiler_params=pltpu.CompilerParams(needs_layout_passes=False),
        out_type=jax.ShapeDtypeStruct((H * E,), jnp.float32),
        scratch_types=[
            pltpu.VMEM((H * N,), jnp.float32),
            pltpu.VMEM((_CHUNK,), jnp.int32),
            pltpu.VMEM((H * _CHUNK,), jnp.float32),
        ],
    )
    def k(ex_hbm, den_hbm, dst_hbm, al_hbm, dtab, dstv, exv):
        wid = lax.axis_index("s") * 2 + lax.axis_index("c")
        ebase = wid * epc
        pltpu.sync_copy(den_hbm, dtab)

        def chunk_body(ci, _):
            eoff = ebase + ci * _CHUNK
            pltpu.sync_copy(dst_hbm.at[pl.ds(eoff, _CHUNK)], dstv)
            for h in range(H):
                pltpu.sync_copy(ex_hbm.at[pl.ds(h * E + eoff, _CHUNK)],
                                exv.at[pl.ds(h * _CHUNK, _CHUNK)])

            def group(b, _g):
                d16 = dstv[pl.ds(b * 16, 16)]
                for h in range(H):
                    x16 = exv[pl.ds(h * _CHUNK + b * 16, 16)]
                    dn16 = plsc.load_gather(dtab, [d16 + h * N])
                    exv[pl.ds(h * _CHUNK + b * 16, 16)] = (
                        x16 / (dn16 + 1e-9) * (1.0 / H))
                return _g
            lax.fori_loop(0, _CHUNK // 16, group, None)
            for h in range(H):
                pltpu.sync_copy(exv.at[pl.ds(h * _CHUNK, _CHUNK)],
                                al_hbm.at[pl.ds(h * E + eoff, _CHUNK)])
            return _
        lax.fori_loop(0, nchunks, chunk_body, None)

    return k(ex, denom, dst_a)


def _sc_pass2(wh_flat, alpha, edge_index, H, DH):
    epc = E // 16
    nchunks = epc // _CHUNK
    mesh = plsc.VectorSubcoreMesh(core_axis_name="c", subcore_axis_name="s")

    @functools.partial(
        pl.kernel, mesh=mesh,
        compiler_params=pltpu.CompilerParams(needs_layout_passes=False),
        out_type=jax.ShapeDtypeStruct((2, _NPAD, DH), jnp.float32),
        scratch_types=[
            pltpu.VMEM((_CHUNK,), jnp.int32),
            pltpu.VMEM((_CHUNK,), jnp.int32),
            pltpu.VMEM((H * _CHUNK,), jnp.float32),
            pltpu.VMEM((16 * H,), jnp.int32),
            pltpu.VMEM((16 * H, DH), jnp.float32),
            pltpu.VMEM((80, DH), jnp.float32),
            pltpu.VMEM((80,), jnp.int32),
            pltpu.VMEM((8, DH), jnp.float32),
            pltpu.VMEM_SHARED((_NPAD, DH), jnp.float32),
            pltpu.SemaphoreType.DMA,
        ],
    )
    def k2(wh_hbm, al_hbm, src_hbm, dst_hbm, out_hbm,
           srcv, dstv, exv, idxb, rows, stage, ridx, zbuf, aggs, sem):
        cid = lax.axis_index("c")
        sid = lax.axis_index("s")
        zero16 = jnp.zeros((16,), jnp.float32)
        for r in range(8):
            for c in range(DH // 16):
                zbuf[r, pl.ds(c * 16, 16)] = zero16

        def zrow(i, _):
            pltpu.sync_copy(zbuf, aggs.at[pl.ds(sid * (_NPAD // 16) + i * 8, 8)])
            return _
        lax.fori_loop(0, _NPAD // 16 // 8, zrow, None)
        plsc.subcore_barrier()

        nlo = cid * _NHALF

        def chunk_body(ci, _):
            eoff = sid * epc + ci * _CHUNK
            pltpu.sync_copy(src_hbm.at[pl.ds(eoff, _CHUNK)], srcv)
            pltpu.sync_copy(dst_hbm.at[pl.ds(eoff, _CHUNK)], dstv)
            for h in range(H):
                pltpu.sync_copy(al_hbm.at[pl.ds(h * E + eoff, _CHUNK)],
                                exv.at[pl.ds(h * _CHUNK, _CHUNK)])

            def group(b, _g):
                sb = (b % 5) * 16
                s16 = srcv[pl.ds(b * 16, 16)]
                d16 = dstv[pl.ds(b * 16, 16)]
                for h in range(H):
                    idxb[pl.ds(h * 16, 16)] = s16 * H + h
                cp = pltpu.async_copy(wh_hbm.at[idxb], rows, sem)
                local = d16 - nlo
                fmask = (local < 0) | (local >= _NHALF)
                ridx[pl.ds(sb, 16)] = jnp.where(fmask, _DUMP, local)
                alphas = []
                for h in range(H):
                    a16 = exv[pl.ds(h * _CHUNK + b * 16, 16)]
                    alphas.append(jnp.where(fmask, 0.0, a16))
                cp.wait()
                for e in range(16):
                    als = [alphas[h][e] for h in range(H)]
                    for c in range(DH // 16):
                        acc = rows[e, pl.ds(c * 16, 16)] * als[0]
                        for h in range(1, H):
                            acc = acc + rows[h * 16 + e, pl.ds(c * 16, 16)] * als[h]
                        stage[sb + e, pl.ds(c * 16, 16)] = acc

                @pl.when(b % 5 == 4)
                def _flush():
                    pltpu.sync_copy(stage, aggs.at[ridx], add=True)
                return _g
            lax.fori_loop(0, _CHUNK // 16, group, None)
            return _
        lax.fori_loop(0, nchunks, chunk_body, None)
        plsc.subcore_barrier()
        rows_per_tile = _NPAD // 16
        pltpu.sync_copy(aggs.at[pl.ds(sid * rows_per_tile, rows_per_tile)],
                        out_hbm.at[cid, pl.ds(sid * rows_per_tile, rows_per_tile)])

    src_a = edge_index[0].astype(jnp.int32)
    dst_a = edge_index[1].astype(jnp.int32)
    return k2(wh_flat, alpha, src_a, dst_a)


def _elu_body(x_ref, o_ref):
    x = x_ref[...]
    o_ref[...] = jnp.where(x > 0, x, jnp.exp(x) - 1.0)


def _elu(x):
    dh = x.shape[1]
    return pl.pallas_call(
        _elu_body,
        grid=(N // TN,),
        in_specs=[pl.BlockSpec((TN, dh), lambda i: (i, 0))],
        out_specs=pl.BlockSpec((TN, dh), lambda i: (i, 0)),
        out_shape=jax.ShapeDtypeStruct((N, dh), jnp.float32),
    )(x)


def _edge_phase(Wh, ea, edge_index, H, DH):
    ea_t = ea.T
    ex, pd = _sc_pass1(ea_t, edge_index, H)
    denom = _psum(pd.reshape(32, H * N)).reshape(H * N)
    alpha = _sc_alpha(ex, denom, edge_index[1].astype(jnp.int32), H)
    src = edge_index[0]
    dst = edge_index[1]
    al = alpha.reshape(H, E).T
    msg = (Wh[src].reshape(E, H, DH) * al[:, :, None]).sum(axis=1)
    agg = jax.ops.segment_sum(msg, dst, num_segments=N)
    return _elu(agg)


def kernel(h, edge_index, W1, a1s, a1d, W2, a2s, a2d, W3, a3s, a3d, Wi, Whh, bi, bh, W4, b4):
    src = edge_index[0]
    dst = edge_index[1]

    def acat(a_s, a_d):
        H, DH = a_s.shape
        eye = jnp.eye(H, dtype=jnp.float32)
        As = (a_s[:, None, :] * eye[:, :, None]).transpose(0, 2, 1).reshape(H * DH, H)
        Ad = (a_d[:, None, :] * eye[:, :, None]).transpose(0, 2, 1).reshape(H * DH, H)
        return jnp.concatenate([As, Ad], axis=1)

    H1, DIN, DH = W1.shape
    Wc1 = W1.transpose(1, 0, 2).reshape(DIN, H1 * DH)
    Wh1, ea1 = _proj(h, Wc1, acat(a1s, a1d))
    h1 = _edge_phase(Wh1, ea1, edge_index, H1, DH)

    H2 = W2.shape[0]
    Wc2 = W2.transpose(1, 0, 2).reshape(DH, H2 * DH)
    Wh2, ea2 = _proj(h1, Wc2, acat(a2s, a2d))
    a2 = _edge_phase(Wh2, ea2, edge_index, H2, DH)
    h2 = _gru(a2, h1, Wi, Whh, bi, bh)

    H3 = W3.shape[0]
    Wc3 = W3.transpose(1, 0, 2).reshape(DH, H3 * DH)
    Wh3, ea3 = _proj(h2, Wc3, acat(a3s, a3d))
    a3 = _edge_phase(Wh3, ea3, edge_index, H3, DH)
    h3 = _gru(a3, h2, Wi, Whh, bi, bh)

    return _final(h3, W4, b4)
